# baseline (speedup 1.0000x reference)
import functools
import math

import jax
import jax.numpy as jnp
from jax import lax
from jax.experimental import pallas as pl
from jax.experimental.pallas import tpu as pltpu
from jax.experimental.pallas import tpu_sc as plsc

NC = 2
NS = 16
LANES = 16

_MESH = dict(core_axis_name="c", subcore_axis_name="s")


def _fill_zero_2d(ref, nrows):
    def row(i, _):
        for k in range(8):
            ref[i, pl.ds(k * LANES, LANES)] = jnp.zeros((LANES,), jnp.float32)
        return 0
    lax.fori_loop(0, nrows, row, 0)


def _fill_const_1d(ref, n, val):
    def body(i, _):
        ref[pl.ds(i * LANES, LANES)] = jnp.full((LANES,), val, jnp.float32)
        return 0
    lax.fori_loop(0, n // LANES, body, 0)


def _make_deg_kernel(n_pad, chunks, b):
    rows_per_tile = n_pad // NS

    @functools.partial(
        pl.kernel,
        out_type=jax.ShapeDtypeStruct((NC, n_pad), jnp.float32),
        mesh=plsc.VectorSubcoreMesh(**_MESH),
        scratch_types=[
            pltpu.VMEM((chunks, b), jnp.int32),
            pltpu.VMEM((48,), jnp.float32),
            pltpu.VMEM((rows_per_tile,), jnp.float32),
            pltpu.VMEM_SHARED((n_pad,), jnp.float32),
        ],
    )
    def deg_kernel(dst_hbm, out_hbm, idx_v, ones_v, zer_v, deg_sh):
        cid = lax.axis_index("c")
        sid = lax.axis_index("s")
        pltpu.sync_copy(dst_hbm.at[cid, sid], idx_v)
        _fill_const_1d(ones_v, 48, 1.0)
        _fill_const_1d(zer_v, rows_per_tile, 0.0)
        pltpu.sync_copy(zer_v, deg_sh.at[pl.ds(sid * rows_per_tile,
                                               rows_per_tile)])
        plsc.subcore_barrier()

        def body(j, _):
            pltpu.sync_copy(ones_v.at[pl.ds(0, b)], deg_sh.at[idx_v.at[j]],
                            add=True)
            return 0
        lax.fori_loop(0, chunks, body, 0)
        plsc.subcore_barrier()
        pltpu.sync_copy(deg_sh.at[pl.ds(sid * rows_per_tile, rows_per_tile)],
                        out_hbm.at[cid, pl.ds(sid * rows_per_tile,
                                              rows_per_tile)])

    return deg_kernel


def _make_prop_kernel(n, groups, cg, b):
    rows_per_tile = n // NS
    assert cg >= 5 and cg % 3 == 1

    @functools.partial(
        pl.kernel,
        out_type=jax.ShapeDtypeStruct((NC, n, 128), jnp.float32),
        mesh=plsc.VectorSubcoreMesh(**_MESH),
        scratch_types=[
            pltpu.VMEM((cg, b), jnp.int32),
            pltpu.VMEM((cg, b), jnp.int32),
            pltpu.VMEM((b, 256), jnp.float32),
            pltpu.VMEM((b, 256), jnp.float32),
            pltpu.VMEM((b, 256), jnp.float32),
            pltpu.VMEM_SHARED((n, 128), jnp.float32),
            pltpu.SemaphoreType.DMA,
            pltpu.SemaphoreType.DMA,
            pltpu.SemaphoreType.DMA,
        ],
    )
    def prop_kernel(h_hbm, src_hbm, dst_hbm, out_hbm,
                    src_v, dst_v, rows0, rows1, rows2, acc_sh,
                    gsem0, gsem1, gsem2):
        cid = lax.axis_index("c")
        sid = lax.axis_index("s")

        plsc.subcore_barrier()

        def group(g, _):
            pltpu.sync_copy(src_hbm.at[cid, sid, g], src_v)
            pltpu.sync_copy(dst_hbm.at[sid, g], dst_v)
            pltpu.async_copy(h_hbm.at[src_v.at[0]], rows0, gsem0)
            pltpu.async_copy(h_hbm.at[src_v.at[1]], rows1, gsem1)

            def body(m, _):
                j = 3 * m
                pltpu.make_async_copy(h_hbm.at[src_v.at[j]], rows0,
                                      gsem0).wait()
                pltpu.async_copy(h_hbm.at[src_v.at[j + 2]], rows2, gsem2)
                pass
                pltpu.make_async_copy(h_hbm.at[src_v.at[j + 1]], rows1,
                                      gsem1).wait()
                pltpu.async_copy(h_hbm.at[src_v.at[j + 3]], rows0, gsem0)
                pass
                pltpu.make_async_copy(h_hbm.at[src_v.at[j + 2]], rows2,
                                      gsem2).wait()
                pltpu.async_copy(h_hbm.at[src_v.at[j + 4]], rows1, gsem1)
                pass
                return 0
            lax.fori_loop(0, (cg - 4) // 3, body, 0)

            pltpu.make_async_copy(h_hbm.at[src_v.at[cg - 4]], rows0,
                                  gsem0).wait()
            pltpu.async_copy(h_hbm.at[src_v.at[cg - 2]], rows2, gsem2)
            pass
            pltpu.make_async_copy(h_hbm.at[src_v.at[cg - 3]], rows1,
                                  gsem1).wait()
            pltpu.async_copy(h_hbm.at[src_v.at[cg - 1]], rows0, gsem0)
            pass
            pltpu.make_async_copy(h_hbm.at[src_v.at[cg - 2]], rows2,
                                  gsem2).wait()
            pass
            pltpu.make_async_copy(h_hbm.at[src_v.at[cg - 1]], rows0,
                                  gsem0).wait()
            pass
            return 0
        lax.fori_loop(0, groups, group, 0)

        plsc.subcore_barrier()
        wrows = (rows_per_tile // 8) * 8
        last = n - wrows * (NS - 1)

        @pl.when(sid < NS - 1)
        def _():
            pltpu.sync_copy(acc_sh.at[pl.ds(sid * wrows, wrows)],
                            out_hbm.at[cid, pl.ds(sid * wrows, wrows)])

        @pl.when(sid == NS - 1)
        def _():
            pltpu.sync_copy(acc_sh.at[pl.ds((NS - 1) * wrows, last)],
                            out_hbm.at[cid, pl.ds((NS - 1) * wrows, last)])

    return prop_kernel


def _prep_tc(degT, feat, n, bn):
    grid = n // bn

    def body(deg_ref, feat_ref, dinv_ref, fs_ref, h_ref):
        d = deg_ref[:, 0:1] + deg_ref[:, 1:2]
        dinv = lax.rsqrt(jnp.maximum(d, 1.0))
        dinv_b = jnp.broadcast_to(dinv, (bn, 128))
        dinv_ref[...] = dinv_b
        f = feat_ref[...]
        f0 = f[:, :128]
        f1 = f[:, 128:]
        fs_ref[0] = f0
        fs_ref[1] = f1
        h_ref[0] = f0 * dinv_b
        h_ref[1] = f1 * dinv_b

    return pl.pallas_call(
        body,
        grid=(grid,),
        in_specs=[
            pl.BlockSpec((bn, 2), lambda i: (i, 0)),
            pl.BlockSpec((bn, 256), lambda i: (i, 0)),
        ],
        out_specs=[
            pl.BlockSpec((bn, 128), lambda i: (i, 0)),
            pl.BlockSpec((2, bn, 128), lambda i: (0, i, 0)),
            pl.BlockSpec((2, bn, 128), lambda i: (0, i, 0)),
        ],
        out_shape=[
            jax.ShapeDtypeStruct((n, 128), jnp.float32),
            jax.ShapeDtypeStruct((2, n, 128), jnp.float32),
            jax.ShapeDtypeStruct((2, n, 128), jnp.float32),
        ],
    )(degT, feat)


def _hop_tc(f, p, dinv_r, n, bn):
    grid = n // bn

    def body(f_ref, p_ref, dinv_ref, fn_ref, hn_ref):
        dv = dinv_ref[...]
        for c in range(2):
            fn = f_ref[c] - p_ref[c] * dv
            fn_ref[c] = fn
            hn_ref[c] = fn * dv

    return pl.pallas_call(
        body,
        grid=(grid,),
        in_specs=[
            pl.BlockSpec((2, bn, 128), lambda i: (0, i, 0)),
            pl.BlockSpec((2, bn, 128), lambda i: (0, i, 0)),
            pl.BlockSpec((bn, 128), lambda i: (i, 0)),
        ],
        out_specs=[
            pl.BlockSpec((2, bn, 128), lambda i: (0, i, 0)),
            pl.BlockSpec((2, bn, 128), lambda i: (0, i, 0)),
        ],
        out_shape=[
            jax.ShapeDtypeStruct((2, n, 128), jnp.float32),
            jax.ShapeDtypeStruct((2, n, 128), jnp.float32),
        ],
    )(f, p, dinv_r)


def _attn_tc(feat, f1, f2, f3, WqT, bq2, Wk, WvT, bv2, n, bn, d, qk):
    grid = n // bn
    scale = 1.0 / math.sqrt(d)

    def body(feat_ref, f1_ref, f2_ref, f3_ref, wqt_ref, bq_ref, wk_ref,
             wvt_ref, bv_ref, out_ref):
        f = feat_ref[...]
        q = jnp.dot(f, wqt_ref[...],
                    preferred_element_type=jnp.float32) + bq_ref[...]
        qw = jnp.dot(q, wk_ref[...], preferred_element_type=jnp.float32)
        f1 = jnp.concatenate([f1_ref[0], f1_ref[1]], axis=1)
        f2 = jnp.concatenate([f2_ref[0], f2_ref[1]], axis=1)
        f3 = jnp.concatenate([f3_ref[0], f3_ref[1]], axis=1)
        a = 2.0 * f - f1
        s0 = jnp.sum(qw * a, axis=1, keepdims=True) * scale
        s1 = jnp.sum(qw * f1, axis=1, keepdims=True) * scale
        s2 = jnp.sum(qw * f2, axis=1, keepdims=True) * scale
        s3 = jnp.sum(qw * f3, axis=1, keepdims=True) * scale
        m = jnp.maximum(jnp.maximum(s0, s1), jnp.maximum(s2, s3))
        e0 = jnp.exp(s0 - m)
        e1 = jnp.exp(s1 - m)
        e2 = jnp.exp(s2 - m)
        e3 = jnp.exp(s3 - m)
        r = 1.0 / (e0 + e1 + e2 + e3)
        mix = (e0 * r) * a + (e1 * r) * f1 + (e2 * r) * f2 + (e3 * r) * f3
        out_ref[...] = jnp.dot(mix, wvt_ref[...],
                               preferred_element_type=jnp.float32) + bv_ref[...]

    wspec = lambda shape: pl.BlockSpec(shape, lambda i: tuple(0 for _ in shape))
    return pl.pallas_call(
        body,
        grid=(grid,),
        in_specs=[
            pl.BlockSpec((bn, 256), lambda i: (i, 0)),
            pl.BlockSpec((2, bn, 128), lambda i: (0, i, 0)),
            pl.BlockSpec((2, bn, 128), lambda i: (0, i, 0)),
            pl.BlockSpec((2, bn, 128), lambda i: (0, i, 0)),
            wspec((256, qk)),
            wspec((1, qk)),
            wspec((qk, 256)),
            wspec((256, 256)),
            wspec((1, 256)),
        ],
        out_specs=pl.BlockSpec((bn, 256), lambda i: (i, 0)),
        out_shape=jax.ShapeDtypeStruct((n, 256), jnp.float32),
    )(feat, f1, f2, f3, WqT, bq2, Wk, WvT, bv2)


def kernel(feat, edge_index, Wq, bq, Wk, bk, Wv, bv):
    n, d = feat.shape
    e = edge_index.shape[1]
    qk = Wq.shape[0]
    assert d == 256 and n % NS == 0

    src = edge_index[0]
    dst = edge_index[1]

    ec = e // NC
    et = ec // NS
    bd = 40
    assert et % bd == 0
    dst_deg = dst.reshape(NC, NS, et // bd, bd)
    n_pad = ((n + 16 * NS - 1) // (16 * NS)) * (16 * NS)
    deg2 = _make_deg_kernel(n_pad, et // bd, bd)(dst_deg)
    degT = deg2.T[:n]

    bn = 1000
    dinv_r, feat_s, h1 = _prep_tc(degT, feat, n, bn)

    bp = 40
    groups, cg = 10, 25
    ept = e // NS
    assert ept == groups * cg * bp
    src2 = jnp.broadcast_to(src.reshape(1, NS, groups, cg, bp),
                            (NC, NS, groups, cg, bp))
    dst_p = dst.reshape(NS, groups, cg, bp)
    prop = _make_prop_kernel(n, groups, cg, bp)

    p1 = prop(h1.transpose(1, 0, 2).reshape(n, 256), src2, dst_p)
    f1, h2 = _hop_tc(feat_s, p1, dinv_r, n, bn)
    p2 = prop(h2.transpose(1, 0, 2).reshape(n, 256), src2, dst_p)
    f2, h3 = _hop_tc(f1, p2, dinv_r, n, bn)
    p3 = prop(h3.transpose(1, 0, 2).reshape(n, 256), src2, dst_p)
    f3, _ = _hop_tc(f2, p3, dinv_r, n, bn)

    out = _attn_tc(feat, f1, f2, f3, Wq.T, bq.reshape(1, qk), Wk, Wv.T,
                   bv.reshape(1, d), n, bn, d, qk)
    return out


# speedup vs baseline: 1.7228x; 1.7228x over previous
import functools
import math

import jax
import jax.numpy as jnp
from jax import lax
from jax.experimental import pallas as pl
from jax.experimental.pallas import tpu as pltpu
from jax.experimental.pallas import tpu_sc as plsc

NC = 2
NS = 16
LANES = 16

_MESH = dict(core_axis_name="c", subcore_axis_name="s")


def _fill_zero_2d(ref, nrows):
    def row(i, _):
        for k in range(8):
            ref[i, pl.ds(k * LANES, LANES)] = jnp.zeros((LANES,), jnp.float32)
        return 0
    lax.fori_loop(0, nrows, row, 0)


def _fill_const_1d(ref, n, val):
    def body(i, _):
        ref[pl.ds(i * LANES, LANES)] = jnp.full((LANES,), val, jnp.float32)
        return 0
    lax.fori_loop(0, n // LANES, body, 0)


def _make_deg_kernel(n_pad, chunks, b):
    rows_per_tile = n_pad // NS

    @functools.partial(
        pl.kernel,
        out_type=jax.ShapeDtypeStruct((NC, n_pad), jnp.float32),
        mesh=plsc.VectorSubcoreMesh(**_MESH),
        scratch_types=[
            pltpu.VMEM((chunks, b), jnp.int32),
            pltpu.VMEM((48,), jnp.float32),
            pltpu.VMEM((rows_per_tile,), jnp.float32),
            pltpu.VMEM_SHARED((n_pad,), jnp.float32),
        ],
    )
    def deg_kernel(dst_hbm, out_hbm, idx_v, ones_v, zer_v, deg_sh):
        cid = lax.axis_index("c")
        sid = lax.axis_index("s")
        pltpu.sync_copy(dst_hbm.at[cid, sid], idx_v)
        _fill_const_1d(ones_v, 48, 1.0)
        _fill_const_1d(zer_v, rows_per_tile, 0.0)
        pltpu.sync_copy(zer_v, deg_sh.at[pl.ds(sid * rows_per_tile,
                                               rows_per_tile)])
        plsc.subcore_barrier()

        def body(j, _):
            pltpu.sync_copy(ones_v.at[pl.ds(0, b)], deg_sh.at[idx_v.at[j]],
                            add=True)
            return 0
        lax.fori_loop(0, chunks, body, 0)
        plsc.subcore_barrier()
        pltpu.sync_copy(deg_sh.at[pl.ds(sid * rows_per_tile, rows_per_tile)],
                        out_hbm.at[cid, pl.ds(sid * rows_per_tile,
                                              rows_per_tile)])

    return deg_kernel


def _make_prop_kernel(n, groups, cg, b):
    rows_per_tile = n // NS
    assert cg >= 5 and cg % 3 == 1

    @functools.partial(
        pl.kernel,
        out_type=jax.ShapeDtypeStruct((NC, n, 128), jnp.float32),
        mesh=plsc.VectorSubcoreMesh(**_MESH),
        scratch_types=[
            pltpu.VMEM((cg, b), jnp.int32),
            pltpu.VMEM((cg, b), jnp.int32),
            pltpu.VMEM((b, 128), jnp.float32),
            pltpu.VMEM((b, 128), jnp.float32),
            pltpu.VMEM((b, 128), jnp.float32),
            pltpu.VMEM_SHARED((n, 128), jnp.float32),
            pltpu.SemaphoreType.DMA,
            pltpu.SemaphoreType.DMA,
            pltpu.SemaphoreType.DMA,
        ],
    )
    def prop_kernel(h_hbm, src_hbm, dst_hbm, out_hbm,
                    src_v, dst_v, rows0, rows1, rows2, acc_sh,
                    gsem0, gsem1, gsem2):
        cid = lax.axis_index("c")
        sid = lax.axis_index("s")

        _fill_zero_2d(rows0, b)
        wrows = (rows_per_tile // 8) * 8
        last = n - wrows * (NS - 1)

        @pl.when(sid < NS - 1)
        def _():
            for z in range(wrows // b):
                pltpu.sync_copy(rows0,
                                acc_sh.at[pl.ds(sid * wrows + z * b, b)])
            rem = wrows - (wrows // b) * b
            if rem:
                pltpu.sync_copy(
                    rows0.at[pl.ds(0, rem)],
                    acc_sh.at[pl.ds(sid * wrows + (wrows // b) * b, rem)])

        @pl.when(sid == NS - 1)
        def _():
            base15 = (NS - 1) * wrows
            for z in range(last // b):
                pltpu.sync_copy(rows0, acc_sh.at[pl.ds(base15 + z * b, b)])
            rem = last - (last // b) * b
            if rem:
                pltpu.sync_copy(
                    rows0.at[pl.ds(0, rem)],
                    acc_sh.at[pl.ds(base15 + (last // b) * b, rem)])

        plsc.subcore_barrier()

        def group(g, _):
            pltpu.sync_copy(src_hbm.at[cid, sid, g], src_v)
            pltpu.sync_copy(dst_hbm.at[sid, g], dst_v)
            pltpu.async_copy(h_hbm.at[src_v.at[0]], rows0, gsem0)
            pltpu.async_copy(h_hbm.at[src_v.at[1]], rows1, gsem1)

            def body(m, _):
                j = 3 * m
                pltpu.make_async_copy(h_hbm.at[src_v.at[j]], rows0,
                                      gsem0).wait()
                pltpu.async_copy(h_hbm.at[src_v.at[j + 2]], rows2, gsem2)
                pltpu.sync_copy(rows0, acc_sh.at[dst_v.at[j]], add=True)
                pltpu.make_async_copy(h_hbm.at[src_v.at[j + 1]], rows1,
                                      gsem1).wait()
                pltpu.async_copy(h_hbm.at[src_v.at[j + 3]], rows0, gsem0)
                pltpu.sync_copy(rows1, acc_sh.at[dst_v.at[j + 1]], add=True)
                pltpu.make_async_copy(h_hbm.at[src_v.at[j + 2]], rows2,
                                      gsem2).wait()
                pltpu.async_copy(h_hbm.at[src_v.at[j + 4]], rows1, gsem1)
                pltpu.sync_copy(rows2, acc_sh.at[dst_v.at[j + 2]], add=True)
                return 0
            lax.fori_loop(0, (cg - 4) // 3, body, 0)

            pltpu.make_async_copy(h_hbm.at[src_v.at[cg - 4]], rows0,
                                  gsem0).wait()
            pltpu.async_copy(h_hbm.at[src_v.at[cg - 2]], rows2, gsem2)
            pltpu.sync_copy(rows0, acc_sh.at[dst_v.at[cg - 4]], add=True)
            pltpu.make_async_copy(h_hbm.at[src_v.at[cg - 3]], rows1,
                                  gsem1).wait()
            pltpu.async_copy(h_hbm.at[src_v.at[cg - 1]], rows0, gsem0)
            pltpu.sync_copy(rows1, acc_sh.at[dst_v.at[cg - 3]], add=True)
            pltpu.make_async_copy(h_hbm.at[src_v.at[cg - 2]], rows2,
                                  gsem2).wait()
            pltpu.sync_copy(rows2, acc_sh.at[dst_v.at[cg - 2]], add=True)
            pltpu.make_async_copy(h_hbm.at[src_v.at[cg - 1]], rows0,
                                  gsem0).wait()
            pltpu.sync_copy(rows0, acc_sh.at[dst_v.at[cg - 1]], add=True)
            return 0
        lax.fori_loop(0, groups, group, 0)

        plsc.subcore_barrier()
        wrows = (rows_per_tile // 8) * 8
        last = n - wrows * (NS - 1)

        @pl.when(sid < NS - 1)
        def _():
            pltpu.sync_copy(acc_sh.at[pl.ds(sid * wrows, wrows)],
                            out_hbm.at[cid, pl.ds(sid * wrows, wrows)])

        @pl.when(sid == NS - 1)
        def _():
            pltpu.sync_copy(acc_sh.at[pl.ds((NS - 1) * wrows, last)],
                            out_hbm.at[cid, pl.ds((NS - 1) * wrows, last)])

    return prop_kernel


def _dinv_block(deg_ref, bn):
    d = deg_ref[:, 0:1] + deg_ref[:, 1:2]
    dinv = lax.rsqrt(jnp.maximum(d, 1.0))
    return jnp.broadcast_to(dinv, (bn, 128))


def _prep_tc(degT, feat, n, bn):
    grid = n // bn

    def body(deg_ref, feat_ref, h_ref):
        dinv_b = _dinv_block(deg_ref, bn)
        f = feat_ref[...]
        h_ref[0] = f[:, :128] * dinv_b
        h_ref[1] = f[:, 128:] * dinv_b

    return pl.pallas_call(
        body,
        grid=(grid,),
        in_specs=[
            pl.BlockSpec((bn, 2), lambda i: (i, 0)),
            pl.BlockSpec((bn, 256), lambda i: (i, 0)),
        ],
        out_specs=pl.BlockSpec((2, bn, 128), lambda i: (0, i, 0)),
        out_shape=jax.ShapeDtypeStruct((2, n, 128), jnp.float32),
    )(degT, feat)


def _hop_tc(f, p, degT, n, bn, flat_f=False):
    grid = n // bn

    def body(f_ref, p_ref, deg_ref, fn_ref, hn_ref):
        dv = _dinv_block(deg_ref, bn)
        for c in range(2):
            if flat_f:
                fc = f_ref[:, c * 128:(c + 1) * 128]
            else:
                fc = f_ref[c]
            fn = fc - p_ref[c] * dv
            fn_ref[c] = fn
            hn_ref[c] = fn * dv

    f_spec = (pl.BlockSpec((bn, 256), lambda i: (i, 0)) if flat_f
              else pl.BlockSpec((2, bn, 128), lambda i: (0, i, 0)))
    return pl.pallas_call(
        body,
        grid=(grid,),
        in_specs=[
            f_spec,
            pl.BlockSpec((2, bn, 128), lambda i: (0, i, 0)),
            pl.BlockSpec((bn, 2), lambda i: (i, 0)),
        ],
        out_specs=[
            pl.BlockSpec((2, bn, 128), lambda i: (0, i, 0)),
            pl.BlockSpec((2, bn, 128), lambda i: (0, i, 0)),
        ],
        out_shape=[
            jax.ShapeDtypeStruct((2, n, 128), jnp.float32),
            jax.ShapeDtypeStruct((2, n, 128), jnp.float32),
        ],
    )(f, p, degT)


def _attn_tc(feat, f1, f2, p3, degT, WqT, bq2, Wk, WvT, bv2, n, bn, d, qk):
    grid = n // bn
    scale = 1.0 / math.sqrt(d)

    def body(feat_ref, f1_ref, f2_ref, p3_ref, deg_ref, wqt_ref, bq_ref,
             wk_ref, wvt_ref, bv_ref, out_ref):
        f = feat_ref[...]
        q = jnp.dot(f, wqt_ref[...],
                    preferred_element_type=jnp.float32) + bq_ref[...]
        qw = jnp.dot(q, wk_ref[...], preferred_element_type=jnp.float32)
        dv = _dinv_block(deg_ref, bn)
        f1 = jnp.concatenate([f1_ref[0], f1_ref[1]], axis=1)
        f2 = jnp.concatenate([f2_ref[0], f2_ref[1]], axis=1)
        f3 = jnp.concatenate([f2_ref[0] - p3_ref[0] * dv,
                              f2_ref[1] - p3_ref[1] * dv], axis=1)
        a = 2.0 * f - f1
        s0 = jnp.sum(qw * a, axis=1, keepdims=True) * scale
        s1 = jnp.sum(qw * f1, axis=1, keepdims=True) * scale
        s2 = jnp.sum(qw * f2, axis=1, keepdims=True) * scale
        s3 = jnp.sum(qw * f3, axis=1, keepdims=True) * scale
        m = jnp.maximum(jnp.maximum(s0, s1), jnp.maximum(s2, s3))
        e0 = jnp.exp(s0 - m)
        e1 = jnp.exp(s1 - m)
        e2 = jnp.exp(s2 - m)
        e3 = jnp.exp(s3 - m)
        r = 1.0 / (e0 + e1 + e2 + e3)
        mix = (e0 * r) * a + (e1 * r) * f1 + (e2 * r) * f2 + (e3 * r) * f3
        out_ref[...] = jnp.dot(mix, wvt_ref[...],
                               preferred_element_type=jnp.float32) + bv_ref[...]

    wspec = lambda shape: pl.BlockSpec(shape, lambda i: tuple(0 for _ in shape))
    return pl.pallas_call(
        body,
        grid=(grid,),
        in_specs=[
            pl.BlockSpec((bn, 256), lambda i: (i, 0)),
            pl.BlockSpec((2, bn, 128), lambda i: (0, i, 0)),
            pl.BlockSpec((2, bn, 128), lambda i: (0, i, 0)),
            pl.BlockSpec((2, bn, 128), lambda i: (0, i, 0)),
            pl.BlockSpec((bn, 2), lambda i: (i, 0)),
            wspec((256, qk)),
            wspec((1, qk)),
            wspec((qk, 256)),
            wspec((256, 256)),
            wspec((1, 256)),
        ],
        out_specs=pl.BlockSpec((bn, 256), lambda i: (i, 0)),
        out_shape=jax.ShapeDtypeStruct((n, 256), jnp.float32),
    )(feat, f1, f2, p3, degT, WqT, bq2, Wk, WvT, bv2)


def kernel(feat, edge_index, Wq, bq, Wk, bk, Wv, bv):
    n, d = feat.shape
    e = edge_index.shape[1]
    qk = Wq.shape[0]
    assert d == 256 and n % NS == 0

    src = edge_index[0]
    dst = edge_index[1]

    ec = e // NC
    et = ec // NS
    bd = 40
    assert et % bd == 0
    dst_deg = dst.reshape(NC, NS, et // bd, bd)
    n_pad = ((n + 16 * NS - 1) // (16 * NS)) * (16 * NS)
    deg2 = _make_deg_kernel(n_pad, et // bd, bd)(dst_deg)
    degT = deg2.T[:n]

    bn = 1000
    h1 = _prep_tc(degT, feat, n, bn)

    bp = 80
    groups, cg = 5, 25
    ept = e // NS
    assert ept == groups * cg * bp
    src2 = jnp.stack([src, src + n]).reshape(NC, NS, groups, cg, bp)
    dst_p = dst.reshape(NS, groups, cg, bp)
    prop = _make_prop_kernel(n, groups, cg, bp)

    p1 = prop(h1.reshape(2 * n, 128), src2, dst_p)
    f1, h2 = _hop_tc(feat, p1, degT, n, bn, flat_f=True)
    p2 = prop(h2.reshape(2 * n, 128), src2, dst_p)
    f2, h3 = _hop_tc(f1, p2, degT, n, bn)
    p3 = prop(h3.reshape(2 * n, 128), src2, dst_p)

    out = _attn_tc(feat, f1, f2, p3, degT, Wq.T, bq.reshape(1, qk), Wk,
                   Wv.T, bv.reshape(1, d), n, bn, d, qk)
    return out


# speedup vs baseline: 1.8001x; 1.0449x over previous
import functools
import math

import jax
import jax.numpy as jnp
from jax import lax
from jax.experimental import pallas as pl
from jax.experimental.pallas import tpu as pltpu
from jax.experimental.pallas import tpu_sc as plsc

NC = 2
NS = 16
LANES = 16

_MESH = dict(core_axis_name="c", subcore_axis_name="s")


def _fill_zero_2d(ref, nrows):
    def row(i, _):
        for k in range(8):
            ref[i, pl.ds(k * LANES, LANES)] = jnp.zeros((LANES,), jnp.float32)
        return 0
    lax.fori_loop(0, nrows, row, 0)


def _fill_const_1d(ref, n, val):
    def body(i, _):
        ref[pl.ds(i * LANES, LANES)] = jnp.full((LANES,), val, jnp.float32)
        return 0
    lax.fori_loop(0, n // LANES, body, 0)


def _make_deg_kernel(n_pad, chunks, b):
    rows_per_tile = n_pad // NS

    @functools.partial(
        pl.kernel,
        out_type=jax.ShapeDtypeStruct((NC, n_pad), jnp.float32),
        mesh=plsc.VectorSubcoreMesh(**_MESH),
        scratch_types=[
            pltpu.VMEM((chunks, b), jnp.int32),
            pltpu.VMEM((48,), jnp.float32),
            pltpu.VMEM((rows_per_tile,), jnp.float32),
            pltpu.VMEM_SHARED((n_pad,), jnp.float32),
        ],
    )
    def deg_kernel(dst_hbm, out_hbm, idx_v, ones_v, zer_v, deg_sh):
        cid = lax.axis_index("c")
        sid = lax.axis_index("s")
        pltpu.sync_copy(dst_hbm.at[cid, sid], idx_v)
        _fill_const_1d(ones_v, 48, 1.0)
        _fill_const_1d(zer_v, rows_per_tile, 0.0)
        pltpu.sync_copy(zer_v, deg_sh.at[pl.ds(sid * rows_per_tile,
                                               rows_per_tile)])
        plsc.subcore_barrier()

        def body(j, _):
            pltpu.sync_copy(ones_v.at[pl.ds(0, b)], deg_sh.at[idx_v.at[j]],
                            add=True)
            return 0
        lax.fori_loop(0, chunks, body, 0)
        plsc.subcore_barrier()
        pltpu.sync_copy(deg_sh.at[pl.ds(sid * rows_per_tile, rows_per_tile)],
                        out_hbm.at[cid, pl.ds(sid * rows_per_tile,
                                              rows_per_tile)])

    return deg_kernel


def _make_prop_kernel(n, groups, cg, b):
    rows_per_tile = n // NS
    assert cg >= 5 and cg % 3 == 1

    @functools.partial(
        pl.kernel,
        out_type=jax.ShapeDtypeStruct((NC, n, 128), jnp.float32),
        mesh=plsc.VectorSubcoreMesh(**_MESH),
        scratch_types=[
            pltpu.VMEM((cg, b), jnp.int32),
            pltpu.VMEM((cg, b), jnp.int32),
            pltpu.VMEM((b, 128), jnp.float32),
            pltpu.VMEM((b, 128), jnp.float32),
            pltpu.VMEM((b, 128), jnp.float32),
            pltpu.VMEM_SHARED((n, 128), jnp.float32),
            pltpu.SemaphoreType.DMA,
            pltpu.SemaphoreType.DMA,
            pltpu.SemaphoreType.DMA,
        ],
    )
    def prop_kernel(h_hbm, src_hbm, dst_hbm, out_hbm,
                    src_v, dst_v, rows0, rows1, rows2, acc_sh,
                    gsem0, gsem1, gsem2):
        cid = lax.axis_index("c")
        sid = lax.axis_index("s")

        _fill_zero_2d(rows0, b)
        wrows = (rows_per_tile // 8) * 8
        last = n - wrows * (NS - 1)

        @pl.when(sid < NS - 1)
        def _():
            for z in range(wrows // b):
                pltpu.sync_copy(rows0,
                                acc_sh.at[pl.ds(sid * wrows + z * b, b)])
            rem = wrows - (wrows // b) * b
            if rem:
                pltpu.sync_copy(
                    rows0.at[pl.ds(0, rem)],
                    acc_sh.at[pl.ds(sid * wrows + (wrows // b) * b, rem)])

        @pl.when(sid == NS - 1)
        def _():
            base15 = (NS - 1) * wrows
            for z in range(last // b):
                pltpu.sync_copy(rows0, acc_sh.at[pl.ds(base15 + z * b, b)])
            rem = last - (last // b) * b
            if rem:
                pltpu.sync_copy(
                    rows0.at[pl.ds(0, rem)],
                    acc_sh.at[pl.ds(base15 + (last // b) * b, rem)])

        plsc.subcore_barrier()

        def group(g, _):
            pltpu.sync_copy(src_hbm.at[cid, sid, g], src_v)
            pltpu.sync_copy(dst_hbm.at[sid, g], dst_v)
            pltpu.async_copy(h_hbm.at[src_v.at[0]], rows0, gsem0)
            pltpu.async_copy(h_hbm.at[src_v.at[1]], rows1, gsem1)

            def body(m, _):
                j = 3 * m
                pltpu.make_async_copy(h_hbm.at[src_v.at[j]], rows0,
                                      gsem0).wait()
                pltpu.async_copy(h_hbm.at[src_v.at[j + 2]], rows2, gsem2)
                pltpu.sync_copy(rows0, acc_sh.at[dst_v.at[j]], add=True)
                pltpu.make_async_copy(h_hbm.at[src_v.at[j + 1]], rows1,
                                      gsem1).wait()
                pltpu.async_copy(h_hbm.at[src_v.at[j + 3]], rows0, gsem0)
                pltpu.sync_copy(rows1, acc_sh.at[dst_v.at[j + 1]], add=True)
                pltpu.make_async_copy(h_hbm.at[src_v.at[j + 2]], rows2,
                                      gsem2).wait()
                pltpu.async_copy(h_hbm.at[src_v.at[j + 4]], rows1, gsem1)
                pltpu.sync_copy(rows2, acc_sh.at[dst_v.at[j + 2]], add=True)
                return 0
            lax.fori_loop(0, (cg - 4) // 3, body, 0)

            pltpu.make_async_copy(h_hbm.at[src_v.at[cg - 4]], rows0,
                                  gsem0).wait()
            pltpu.async_copy(h_hbm.at[src_v.at[cg - 2]], rows2, gsem2)
            pltpu.sync_copy(rows0, acc_sh.at[dst_v.at[cg - 4]], add=True)
            pltpu.make_async_copy(h_hbm.at[src_v.at[cg - 3]], rows1,
                                  gsem1).wait()
            pltpu.async_copy(h_hbm.at[src_v.at[cg - 1]], rows0, gsem0)
            pltpu.sync_copy(rows1, acc_sh.at[dst_v.at[cg - 3]], add=True)
            pltpu.make_async_copy(h_hbm.at[src_v.at[cg - 2]], rows2,
                                  gsem2).wait()
            pltpu.sync_copy(rows2, acc_sh.at[dst_v.at[cg - 2]], add=True)
            pltpu.make_async_copy(h_hbm.at[src_v.at[cg - 1]], rows0,
                                  gsem0).wait()
            pltpu.sync_copy(rows0, acc_sh.at[dst_v.at[cg - 1]], add=True)
            return 0
        lax.fori_loop(0, groups, group, 0)

        plsc.subcore_barrier()
        wrows = (rows_per_tile // 8) * 8
        last = n - wrows * (NS - 1)

        @pl.when(sid < NS - 1)
        def _():
            pltpu.sync_copy(acc_sh.at[pl.ds(sid * wrows, wrows)],
                            out_hbm.at[cid, pl.ds(sid * wrows, wrows)])

        @pl.when(sid == NS - 1)
        def _():
            pltpu.sync_copy(acc_sh.at[pl.ds((NS - 1) * wrows, last)],
                            out_hbm.at[cid, pl.ds((NS - 1) * wrows, last)])

    return prop_kernel


def _dinv_block(deg_ref, bn):
    d = deg_ref[:, 0:1] + deg_ref[:, 1:2]
    dinv = lax.rsqrt(jnp.maximum(d, 1.0))
    return jnp.broadcast_to(dinv, (bn, 128))


def _prep_tc(degT, feat, n, bn):
    grid = n // bn

    def body(deg_ref, feat_ref, h_ref):
        dinv_b = _dinv_block(deg_ref, bn)
        f = feat_ref[...]
        h_ref[0] = f[:, :128] * dinv_b
        h_ref[1] = f[:, 128:] * dinv_b

    return pl.pallas_call(
        body,
        grid=(grid,),
        in_specs=[
            pl.BlockSpec((bn, 2), lambda i: (i, 0)),
            pl.BlockSpec((bn, 256), lambda i: (i, 0)),
        ],
        out_specs=pl.BlockSpec((2, bn, 128), lambda i: (0, i, 0)),
        out_shape=jax.ShapeDtypeStruct((2, n, 128), jnp.float32),
    )(degT, feat)


def _hop_tc(f, p, degT, n, bn, flat_f=False):
    grid = n // bn

    def body(f_ref, p_ref, deg_ref, fn_ref, hn_ref):
        dv = _dinv_block(deg_ref, bn)
        for c in range(2):
            if flat_f:
                fc = f_ref[:, c * 128:(c + 1) * 128]
            else:
                fc = f_ref[c]
            fn = fc - p_ref[c] * dv
            fn_ref[c] = fn
            hn_ref[c] = fn * dv

    f_spec = (pl.BlockSpec((bn, 256), lambda i: (i, 0)) if flat_f
              else pl.BlockSpec((2, bn, 128), lambda i: (0, i, 0)))
    return pl.pallas_call(
        body,
        grid=(grid,),
        in_specs=[
            f_spec,
            pl.BlockSpec((2, bn, 128), lambda i: (0, i, 0)),
            pl.BlockSpec((bn, 2), lambda i: (i, 0)),
        ],
        out_specs=[
            pl.BlockSpec((2, bn, 128), lambda i: (0, i, 0)),
            pl.BlockSpec((2, bn, 128), lambda i: (0, i, 0)),
        ],
        out_shape=[
            jax.ShapeDtypeStruct((2, n, 128), jnp.float32),
            jax.ShapeDtypeStruct((2, n, 128), jnp.float32),
        ],
    )(f, p, degT)


def _attn_tc(feat, f1, f2, p3, degT, WqT, bq2, Wk, WvT, bv2, n, bn, d, qk):
    grid = n // bn
    scale = 1.0 / math.sqrt(d)

    def body(feat_ref, f1_ref, f2_ref, p3_ref, deg_ref, wqt_ref, bq_ref,
             wk_ref, wvt_ref, bv_ref, out_ref):
        f = feat_ref[...]
        q = jnp.dot(f, wqt_ref[...],
                    preferred_element_type=jnp.float32) + bq_ref[...]
        qw = jnp.dot(q, wk_ref[...], preferred_element_type=jnp.float32)
        dv = _dinv_block(deg_ref, bn)
        f1 = jnp.concatenate([f1_ref[0], f1_ref[1]], axis=1)
        f2 = jnp.concatenate([f2_ref[0], f2_ref[1]], axis=1)
        f3 = jnp.concatenate([f2_ref[0] - p3_ref[0] * dv,
                              f2_ref[1] - p3_ref[1] * dv], axis=1)
        a = 2.0 * f - f1
        s0 = jnp.sum(qw * a, axis=1, keepdims=True) * scale
        s1 = jnp.sum(qw * f1, axis=1, keepdims=True) * scale
        s2 = jnp.sum(qw * f2, axis=1, keepdims=True) * scale
        s3 = jnp.sum(qw * f3, axis=1, keepdims=True) * scale
        m = jnp.maximum(jnp.maximum(s0, s1), jnp.maximum(s2, s3))
        e0 = jnp.exp(s0 - m)
        e1 = jnp.exp(s1 - m)
        e2 = jnp.exp(s2 - m)
        e3 = jnp.exp(s3 - m)
        r = 1.0 / (e0 + e1 + e2 + e3)
        mix = (e0 * r) * a + (e1 * r) * f1 + (e2 * r) * f2 + (e3 * r) * f3
        out_ref[...] = jnp.dot(mix, wvt_ref[...],
                               preferred_element_type=jnp.float32) + bv_ref[...]

    wspec = lambda shape: pl.BlockSpec(shape, lambda i: tuple(0 for _ in shape))
    return pl.pallas_call(
        body,
        grid=(grid,),
        in_specs=[
            pl.BlockSpec((bn, 256), lambda i: (i, 0)),
            pl.BlockSpec((2, bn, 128), lambda i: (0, i, 0)),
            pl.BlockSpec((2, bn, 128), lambda i: (0, i, 0)),
            pl.BlockSpec((2, bn, 128), lambda i: (0, i, 0)),
            pl.BlockSpec((bn, 2), lambda i: (i, 0)),
            wspec((256, qk)),
            wspec((1, qk)),
            wspec((qk, 256)),
            wspec((256, 256)),
            wspec((1, 256)),
        ],
        out_specs=pl.BlockSpec((bn, 256), lambda i: (i, 0)),
        out_shape=jax.ShapeDtypeStruct((n, 256), jnp.float32),
    )(feat, f1, f2, p3, degT, WqT, bq2, Wk, WvT, bv2)


def kernel(feat, edge_index, Wq, bq, Wk, bk, Wv, bv):
    n, d = feat.shape
    e = edge_index.shape[1]
    qk = Wq.shape[0]
    assert d == 256 and n % NS == 0

    src = edge_index[0]
    dst = edge_index[1]

    ec = e // NC
    et = ec // NS
    bd = 40
    assert et % bd == 0
    dst_deg = dst.reshape(NC, NS, et // bd, bd)
    n_pad = ((n + 16 * NS - 1) // (16 * NS)) * (16 * NS)
    deg2 = _make_deg_kernel(n_pad, et // bd, bd)(dst_deg)
    degT = deg2.T[:n]

    bn = 1000
    h1 = _prep_tc(degT, feat, n, bn)

    bp = 100
    groups, cg = 4, 25
    ept = e // NS
    assert ept == groups * cg * bp
    src2 = jnp.stack([src, src + n]).reshape(NC, NS, groups, cg, bp)
    dst_p = dst.reshape(NS, groups, cg, bp)
    prop = _make_prop_kernel(n, groups, cg, bp)

    p1 = prop(h1.reshape(2 * n, 128), src2, dst_p)
    f1, h2 = _hop_tc(feat, p1, degT, n, bn, flat_f=True)
    p2 = prop(h2.reshape(2 * n, 128), src2, dst_p)
    f2, h3 = _hop_tc(f1, p2, degT, n, bn)
    p3 = prop(h3.reshape(2 * n, 128), src2, dst_p)

    out = _attn_tc(feat, f1, f2, p3, degT, Wq.T, bq.reshape(1, qk), Wk,
                   Wv.T, bv.reshape(1, d), n, bn, d, qk)
    return out
